# baseline (speedup 1.0000x reference)
import functools

import jax
import jax.numpy as jnp
from jax import lax
from jax.experimental import pallas as pl
from jax.experimental.pallas import tpu as pltpu
from jax.experimental.pallas import tpu_sc as plsc

_L = 16


@functools.lru_cache(maxsize=None)
def _make_gather(n_rows: int, d_model: int):
    info = plsc.get_sparse_core_info()
    nw = info.num_cores * info.num_subcores
    rows_per_w = n_rows // nw
    r = 56
    sizes = [r] * (rows_per_w // r)
    if rows_per_w % r:
        sizes.append(rows_per_w % r)
    offs = [sum(sizes[:i]) for i in range(len(sizes))]
    n_chunks = len(sizes)

    mesh = plsc.VectorSubcoreMesh(core_axis_name="c", subcore_axis_name="s")

    @functools.partial(
        pl.kernel,
        mesh=mesh,
        out_type=jax.ShapeDtypeStruct((n_rows, d_model), jnp.float32),
        scratch_types=[
            pltpu.VMEM((_L,), jnp.int32),
            pltpu.VMEM((32,), jnp.int32),
            pltpu.VMEM((32,), jnp.int32),
            pltpu.VMEM((2, r, d_model), jnp.float32),
            pltpu.SemaphoreType.DMA,
        ],
    )
    def k(seq_hbm, table_hbm, out_hbm, seq_v, idx0_v, idx1_v, rows_v, sem):
        wid = lax.axis_index("s") * info.num_cores + lax.axis_index("c")
        base = wid * rows_per_w
        pltpu.sync_copy(seq_hbm, seq_v)
        sl = seq_v[...]
        sl_s = sl[0]
        start = lax.rem(base, sl_s)
        fast = jnp.logical_and((start + rows_per_w) <= sl_s,
                               lax.rem(start, 8) == 0)

        @pl.when(fast)
        def _():
            s_al = pl.multiple_of(start, 8)

            def start_lin(c):
                return pltpu.async_copy(
                    table_hbm.at[pl.ds(s_al + offs[c], sizes[c])],
                    rows_v.at[c % 2, pl.ds(0, sizes[c])], sem)

            g = start_lin(0)
            for c in range(n_chunks):
                g_next = start_lin(c + 1) if c + 1 < n_chunks else None
                g.wait()
                pltpu.sync_copy(rows_v.at[c % 2, pl.ds(0, sizes[c])],
                                out_hbm.at[pl.ds(base + offs[c], sizes[c])])
                g = g_next

        @pl.when(jnp.logical_not(fast))
        def _():
            rs = 32
            ns = rows_per_w // rs

            def start_gather(c):
                b = c % 2
                idx_v = idx0_v if b == 0 else idx1_v
                row0 = base + c * rs
                for j in range(rs // _L):
                    idx_v[pl.ds(j * _L, _L)] = lax.rem(
                        (row0 + j * _L) + lax.iota(jnp.int32, _L), sl)
                return pltpu.async_copy(table_hbm.at[idx_v],
                                        rows_v.at[b, pl.ds(0, rs)], sem)

            g = start_gather(0)
            for c in range(ns):
                g_next = start_gather(c + 1) if c + 1 < ns else None
                g.wait()
                pltpu.sync_copy(rows_v.at[c % 2, pl.ds(0, rs)],
                                out_hbm.at[pl.ds(base + c * rs, rs)])
                g = g_next

    return k


def kernel(seq_len, table):
    n_rows, d_model = table.shape
    seq_arr = jnp.full((_L,), seq_len, dtype=jnp.int32)
    return _make_gather(n_rows, d_model)(seq_arr, table)
